# baseline (speedup 1.0000x reference)
import jax
import jax.numpy as jnp
from jax import lax
from jax.experimental import pallas as pl
from jax.experimental.pallas import tpu as pltpu
from jax.experimental.pallas import tpu_sc as plsc

MAXLEN = 2048
VOCAB = 100000
D_MODEL = 768
BATCH = 4

NUM_CORES = 2
NUM_SUBCORES = 16
NW = NUM_CORES * NUM_SUBCORES
ROWS = BATCH * MAXLEN
B_PER_W = ROWS // NW
CHUNK = 64
NCHUNK = B_PER_W // CHUNK
LANES = 16
D_VECS = D_MODEL // LANES


def _emb_kernel(x_hbm, table_hbm, pos_hbm, out_hbm, idx_v, buf, pbuf, sem):
    wid = lax.axis_index("s") * NUM_CORES + lax.axis_index("c")
    base = wid * B_PER_W
    pos_base = base % MAXLEN

    def add_row(r, _):
        for j in range(D_VECS):
            sl = pl.ds(j * LANES, LANES)
            buf[r, sl] = buf[r, sl] + pbuf[r, sl]
        return 0

    for c in range(NCHUNK):
        off = c * CHUNK
        pltpu.sync_copy(x_hbm.at[pl.ds(base + off, CHUNK)], idx_v)
        gather = pltpu.async_copy(table_hbm.at[idx_v], buf, sem)
        pltpu.sync_copy(pos_hbm.at[pl.ds(pos_base + off, CHUNK), :], pbuf)
        gather.wait()
        lax.fori_loop(0, CHUNK, add_row, 0)
        pltpu.sync_copy(buf, out_hbm.at[pl.ds(base + off, CHUNK), :])


def kernel(x, table, pos_enc):
    flat_x = x.reshape(ROWS)
    mesh = plsc.VectorSubcoreMesh(core_axis_name="c", subcore_axis_name="s")
    run = pl.kernel(
        _emb_kernel,
        out_type=jax.ShapeDtypeStruct((ROWS, D_MODEL), jnp.float32),
        mesh=mesh,
        scratch_types=[
            pltpu.VMEM((CHUNK,), jnp.int32),
            pltpu.VMEM((CHUNK, D_MODEL), jnp.float32),
            pltpu.VMEM((CHUNK, D_MODEL), jnp.float32),
            pltpu.SemaphoreType.DMA,
        ],
    )
    out = run(flat_x, table, pos_enc)
    return out.reshape(BATCH, MAXLEN, D_MODEL)


# speedup vs baseline: 1.1932x; 1.1932x over previous
import jax
import jax.numpy as jnp
from jax import lax
from jax.experimental import pallas as pl
from jax.experimental.pallas import tpu as pltpu
from jax.experimental.pallas import tpu_sc as plsc

MAXLEN = 2048
VOCAB = 100000
D_MODEL = 768
BATCH = 4

NUM_CORES = 2
NUM_SUBCORES = 16
NW = NUM_CORES * NUM_SUBCORES
ROWS = BATCH * MAXLEN
B_PER_W = ROWS // NW
CHUNK = 32
NCHUNK = B_PER_W // CHUNK
LANES = 16
D_VECS = D_MODEL // LANES


def _emb_kernel(x_hbm, table_hbm, pos_hbm, out_hbm,
                idx_all, buf0, buf1, pbuf0, pbuf1,
                gsem0, gsem1, psem0, psem1, ssem0, ssem1):
    wid = lax.axis_index("s") * NUM_CORES + lax.axis_index("c")
    base = wid * B_PER_W
    pos_base = base % MAXLEN

    bufs = (buf0, buf1)
    pbufs = (pbuf0, pbuf1)
    gsems = (gsem0, gsem1)
    psems = (psem0, psem1)
    ssems = (ssem0, ssem1)

    pltpu.sync_copy(x_hbm.at[pl.ds(base, B_PER_W)], idx_all)

    def issue(c):
        b = c & 1
        g = pltpu.async_copy(
            table_hbm.at[idx_all.at[pl.ds(c * CHUNK, CHUNK)]], bufs[b], gsems[b])
        p = pltpu.async_copy(
            pos_hbm.at[pl.ds(pos_base + c * CHUNK, CHUNK), :], pbufs[b], psems[b])
        return g, p

    pend = {0: issue(0)}
    stores = {}
    for c in range(NCHUNK):
        b = c & 1
        if c + 1 < NCHUNK:
            if c - 1 >= 0:
                stores.pop(c - 1).wait()
            pend[c + 1] = issue(c + 1)
        g, p = pend.pop(c)
        g.wait()
        p.wait()

        buf, pbuf = bufs[b], pbufs[b]

        def add_row(r, _, buf=buf, pbuf=pbuf):
            for j in range(D_VECS):
                sl = pl.ds(j * LANES, LANES)
                buf[r, sl] = buf[r, sl] + pbuf[r, sl]
            return 0

        lax.fori_loop(0, CHUNK, add_row, 0)
        stores[c] = pltpu.async_copy(
            buf, out_hbm.at[pl.ds(base + c * CHUNK, CHUNK), :], ssems[b])
    for s in stores.values():
        s.wait()


def kernel(x, table, pos_enc):
    flat_x = x.reshape(ROWS)
    mesh = plsc.VectorSubcoreMesh(core_axis_name="c", subcore_axis_name="s")
    run = pl.kernel(
        _emb_kernel,
        out_type=jax.ShapeDtypeStruct((ROWS, D_MODEL), jnp.float32),
        mesh=mesh,
        scratch_types=[
            pltpu.VMEM((B_PER_W,), jnp.int32),
            pltpu.VMEM((CHUNK, D_MODEL), jnp.float32),
            pltpu.VMEM((CHUNK, D_MODEL), jnp.float32),
            pltpu.VMEM((CHUNK, D_MODEL), jnp.float32),
            pltpu.VMEM((CHUNK, D_MODEL), jnp.float32),
            pltpu.SemaphoreType.DMA,
            pltpu.SemaphoreType.DMA,
            pltpu.SemaphoreType.DMA,
            pltpu.SemaphoreType.DMA,
            pltpu.SemaphoreType.DMA,
            pltpu.SemaphoreType.DMA,
        ],
    )
    out = run(flat_x, table, pos_enc)
    return out.reshape(BATCH, MAXLEN, D_MODEL)
